# Initial kernel scaffold; baseline (speedup 1.0000x reference)
import jax
import jax.numpy as jnp
from jax.experimental import pallas as pl


def kernel(x, edge_index, W0, b0, W1, b1, W2, b2, W3, b3):
    raise NotImplementedError("write your pallas kernel here")


# speedup vs baseline: 6.2111x; 6.2111x over previous
import functools

import jax
import jax.numpy as jnp
from jax import lax
from jax.experimental import pallas as pl
from jax.experimental.pallas import tpu as pltpu
from jax.experimental.pallas import tpu_sc as plsc

N = 10000
D = 128
NC = 2
NS = 16
NW = NC * NS
CH = 128
BN = 1000


def _mesh():
    return plsc.VectorSubcoreMesh(core_axis_name="c", subcore_axis_name="s")


IG = 16


def _sc_degree(nacc, groups):
    mesh = _mesh()
    rpt = nacc // NS

    @functools.partial(
        pl.kernel,
        mesh=mesh,
        out_type=jax.ShapeDtypeStruct((NC, nacc, D), jnp.float32),
        scratch_types=[
            pltpu.VMEM((IG, CH), jnp.int32),
            pltpu.VMEM((CH, D), jnp.float32),
            pltpu.VMEM_SHARED((nacc, D), jnp.float32),
        ],
    )
    def deg_kernel(idx_hbm, out_hbm, dstv, ones, acc):
        c = lax.axis_index("c")
        s = lax.axis_index("s")
        wid = c * NS + s
        o16 = jnp.ones((16,), jnp.float32)
        z16 = jnp.zeros((16,), jnp.float32)

        def zfill(i, _):
            for j in range(D // 16):
                ones[i, pl.ds(j * 16, 16)] = z16
            return 0

        def zcopy(k, _):
            pltpu.sync_copy(ones, acc.at[pl.ds(s * rpt + k * CH, CH)])
            return 0

        def fill(i, _):
            for j in range(D // 16):
                ones[i, pl.ds(j * 16, 16)] = o16
            return 0

        lax.fori_loop(0, CH, zfill, 0)
        lax.fori_loop(0, rpt // CH, zcopy, 0)
        lax.fori_loop(0, CH, fill, 0)
        plsc.subcore_barrier()

        def body(g, _):
            pltpu.sync_copy(idx_hbm.at[wid, g, 1], dstv)
            for k in range(IG):
                pltpu.sync_copy(ones, acc.at[dstv.at[k]], add=True)
            return 0

        lax.fori_loop(0, groups, body, 0)
        plsc.subcore_barrier()
        pltpu.sync_copy(
            acc.at[pl.ds(s * rpt, rpt)], out_hbm.at[c, pl.ds(s * rpt, rpt)]
        )

    return deg_kernel


def _sc_aggregate(nacc, groups):
    mesh = _mesh()
    rpt = nacc // NS

    @functools.partial(
        pl.kernel,
        mesh=mesh,
        out_type=jax.ShapeDtypeStruct((NC, nacc, D), jnp.float32),
        scratch_types=[
            pltpu.VMEM((2, 2, IG, CH), jnp.int32),
            pltpu.VMEM((2, CH, D), jnp.float32),
            pltpu.VMEM_SHARED((nacc, D), jnp.float32),
            pltpu.SemaphoreType.DMA,
            pltpu.SemaphoreType.DMA,
            pltpu.SemaphoreType.DMA,
        ],
    )
    def agg_kernel(hs_hbm, idx_hbm, out_hbm, ibuf, rbuf, acc, isem, gsem0, gsem1):
        c = lax.axis_index("c")
        s = lax.axis_index("s")
        wid = c * NS + s
        z16 = jnp.zeros((16,), jnp.float32)

        def zfill(i, _):
            for j in range(D // 16):
                rbuf[0, i, pl.ds(j * 16, 16)] = z16
            return 0

        lax.fori_loop(0, CH, zfill, 0)

        def zcopy(k, _):
            pltpu.sync_copy(rbuf.at[0], acc.at[pl.ds(s * rpt + k * CH, CH)])
            return 0

        lax.fori_loop(0, rpt // CH, zcopy, 0)
        plsc.subcore_barrier()

        pltpu.async_copy(idx_hbm.at[wid, 0], ibuf.at[0], isem)
        gsems = (gsem0, gsem1)

        def group_body(g, _):
            p = g % 2
            ib = ibuf.at[p]
            pltpu.make_async_copy(idx_hbm.at[wid, g], ib, isem).wait()

            @pl.when(g + 1 < groups)
            def _():
                pltpu.async_copy(idx_hbm.at[wid, g + 1], ibuf.at[1 - p], isem)

            pltpu.async_copy(hs_hbm.at[ib.at[0, 0]], rbuf.at[0], gsem0)
            for k in range(IG):
                if k + 1 < IG:
                    pltpu.async_copy(
                        hs_hbm.at[ib.at[0, k + 1]], rbuf.at[(k + 1) % 2],
                        gsems[(k + 1) % 2],
                    )
                pltpu.make_async_copy(
                    hs_hbm.at[ib.at[0, k]], rbuf.at[k % 2], gsems[k % 2]
                ).wait()
                pltpu.sync_copy(rbuf.at[k % 2], acc.at[ib.at[1, k]], add=True)
            return 0

        lax.fori_loop(0, groups, group_body, 0)
        plsc.subcore_barrier()
        pltpu.sync_copy(
            acc.at[pl.ds(s * rpt, rpt)], out_hbm.at[c, pl.ds(s * rpt, rpt)]
        )

    return agg_kernel


def _tc_first(x, w, degp):
    n = x.shape[0]
    grid = n // BN

    def body(x_ref, w_ref, dp_ref, dinv_ref, hs_ref):
        deg = 1.0 + dp_ref[0, :, 0] + dp_ref[1, :, 0]
        dinv = lax.rsqrt(deg)[:, None]
        m = jnp.dot(x_ref[...], w_ref[...], preferred_element_type=jnp.float32)
        dinv_ref[...] = jnp.broadcast_to(dinv, (BN, D))
        hs_ref[...] = dinv * m

    return pl.pallas_call(
        body,
        grid=(grid,),
        in_specs=[
            pl.BlockSpec((BN, D), lambda i: (i, 0)),
            pl.BlockSpec((D, D), lambda i: (0, 0)),
            pl.BlockSpec((NC, BN, D), lambda i: (0, i, 0)),
        ],
        out_specs=[
            pl.BlockSpec((BN, D), lambda i: (i, 0)),
            pl.BlockSpec((BN, D), lambda i: (i, 0)),
        ],
        out_shape=[
            jax.ShapeDtypeStruct((n, D), jnp.float32),
            jax.ShapeDtypeStruct((n, D), jnp.float32),
        ],
    )(x, w, degp)


def _tc_combine(sparts, hs, dinv_b, b, identity, w_next, relu):
    n = hs.shape[0]
    grid = n // BN
    have_res = identity is not None
    have_mm = w_next is not None

    def body(*refs):
        i = 0
        sp_ref = refs[i]; i += 1
        hs_ref = refs[i]; i += 1
        dinv_ref = refs[i]; i += 1
        b_ref = refs[i]; i += 1
        id_ref = None
        w_ref = None
        if have_res:
            id_ref = refs[i]; i += 1
        if have_mm:
            w_ref = refs[i]; i += 1
        h_ref = refs[i]; i += 1
        hs2_ref = refs[i] if have_mm else None

        dinv = dinv_ref[...]
        agg = dinv * (sp_ref[0] + sp_ref[1] + hs_ref[...]) + b_ref[...]
        if have_res:
            agg = agg + id_ref[...]
        h = jnp.maximum(agg, 0.0) if relu else agg
        h_ref[...] = h
        if have_mm:
            m = jnp.dot(h, w_ref[...], preferred_element_type=jnp.float32)
            hs2_ref[...] = dinv * m

    in_specs = [
        pl.BlockSpec((NC, BN, D), lambda i: (0, i, 0)),
        pl.BlockSpec((BN, D), lambda i: (i, 0)),
        pl.BlockSpec((BN, D), lambda i: (i, 0)),
        pl.BlockSpec((1, D), lambda i: (0, 0)),
    ]
    args = [sparts, hs, dinv_b, b.reshape(1, D)]
    if have_res:
        in_specs.append(pl.BlockSpec((BN, D), lambda i: (i, 0)))
        args.append(identity)
    if have_mm:
        in_specs.append(pl.BlockSpec((D, D), lambda i: (0, 0)))
        args.append(w_next)

    out_specs = [pl.BlockSpec((BN, D), lambda i: (i, 0))]
    out_shape = [jax.ShapeDtypeStruct((n, D), jnp.float32)]
    if have_mm:
        out_specs.append(pl.BlockSpec((BN, D), lambda i: (i, 0)))
        out_shape.append(jax.ShapeDtypeStruct((n, D), jnp.float32))

    res = pl.pallas_call(
        body,
        grid=(grid,),
        in_specs=in_specs,
        out_specs=out_specs,
        out_shape=out_shape,
    )(*args)
    return res if have_mm else (res[0], None)


def kernel(x, edge_index, W0, b0, W1, b1, W2, b2, W3, b3):
    n, d = x.shape
    e = edge_index.shape[1]
    assert n == N and d == D and n % BN == 0

    nacc = -(-(n + 1) // (NS * CH)) * (NS * CH)
    groups = -(-e // (NW * IG * CH))
    ep = groups * IG * CH * NW

    src = jnp.concatenate([edge_index[0], jnp.zeros((ep - e,), jnp.int32)])
    dst = jnp.concatenate([edge_index[1], jnp.full((ep - e,), n, jnp.int32)])
    src_r = src.reshape(NW, groups, IG, CH)
    dst_r = dst.reshape(NW, groups, IG, CH)
    idx_pack = jnp.stack([src_r, dst_r], axis=2)

    degp = _sc_degree(nacc, groups)(idx_pack)
    dinv_b, hs = _tc_first(x, W0, degp)

    agg = _sc_aggregate(nacc, groups)
    Ws = [W1, W2, W3, None]
    bs = [b0, b1, b2, b3]
    h_prev = None
    for i in range(4):
        sparts = agg(hs, idx_pack)
        identity = h_prev if i in (1, 2) else None
        h, hs_next = _tc_combine(
            sparts, hs, dinv_b, bs[i], identity, Ws[i], relu=(i < 3)
        )
        h_prev = h
        hs = hs_next
    return h_prev


# speedup vs baseline: 16.7262x; 2.6930x over previous
import functools

import jax
import jax.numpy as jnp
from jax import lax
from jax.experimental import pallas as pl
from jax.experimental.pallas import tpu as pltpu
from jax.experimental.pallas import tpu_sc as plsc

N = 10000
D = 128
NC = 2
NS = 16
NW = NC * NS
CH = 128
BN = 1000


def _mesh():
    return plsc.VectorSubcoreMesh(core_axis_name="c", subcore_axis_name="s")


IG = 16


def _sc_degree(nacc, groups):
    mesh = _mesh()
    rpt = nacc // NS

    @functools.partial(
        pl.kernel,
        mesh=mesh,
        out_type=jax.ShapeDtypeStruct((NC, nacc, D), jnp.float32),
        scratch_types=[
            pltpu.VMEM((IG, CH), jnp.int32),
            pltpu.VMEM((CH, D), jnp.float32),
            pltpu.VMEM_SHARED((nacc, D), jnp.float32),
        ],
    )
    def deg_kernel(idx_hbm, out_hbm, dstv, ones, acc):
        c = lax.axis_index("c")
        s = lax.axis_index("s")
        wid = c * NS + s
        o16 = jnp.ones((16,), jnp.float32)
        z16 = jnp.zeros((16,), jnp.float32)

        def zfill(i, _):
            for j in range(D // 16):
                ones[i, pl.ds(j * 16, 16)] = z16
            return 0

        def zcopy(k, _):
            pltpu.sync_copy(ones, acc.at[pl.ds(s * rpt + k * CH, CH)])
            return 0

        def fill(i, _):
            for j in range(D // 16):
                ones[i, pl.ds(j * 16, 16)] = o16
            return 0

        lax.fori_loop(0, CH, zfill, 0)
        lax.fori_loop(0, rpt // CH, zcopy, 0)
        lax.fori_loop(0, CH, fill, 0)
        plsc.subcore_barrier()

        def body(g, _):
            pltpu.sync_copy(idx_hbm.at[wid, g, 1], dstv)
            for k in range(IG):
                pltpu.sync_copy(ones, acc.at[dstv.at[k]], add=True)
            return 0

        lax.fori_loop(0, groups, body, 0)
        plsc.subcore_barrier()
        pltpu.sync_copy(
            acc.at[pl.ds(s * rpt, rpt)], out_hbm.at[c, pl.ds(s * rpt, rpt)]
        )

    return deg_kernel


HD = D // 2


def _sc_aggregate(nacc, groups):
    mesh = _mesh()
    rpt = nacc // NS

    @functools.partial(
        pl.kernel,
        mesh=mesh,
        out_type=jax.ShapeDtypeStruct((NC, nacc, D), jnp.float32),
        compiler_params=pltpu.CompilerParams(use_tc_tiling_on_sc=False),
        scratch_types=[
            pltpu.VMEM((2, 2, IG, CH), jnp.int32),
            pltpu.VMEM((2, CH, HD), jnp.float32),
            pltpu.VMEM_SHARED((nacc, HD), jnp.float32),
            pltpu.VMEM_SHARED((nacc, HD), jnp.float32),
            pltpu.SemaphoreType.DMA,
            pltpu.SemaphoreType.DMA,
            pltpu.SemaphoreType.DMA,
        ],
    )
    def agg_kernel(hs_hbm, idx_hbm, out_hbm, ibuf, rbuf, tbl, acc, isem, gsem0, gsem1):
        c = lax.axis_index("c")
        s = lax.axis_index("s")
        wid = c * NS + s
        z16 = jnp.zeros((16,), jnp.float32)
        gsems = (gsem0, gsem1)

        for p in range(2):
            pltpu.sync_copy(
                hs_hbm.at[pl.ds(s * rpt, rpt), pl.ds(p * HD, HD)],
                tbl.at[pl.ds(s * rpt, rpt)],
            )

            def zfill(i, _):
                for j in range(HD // 16):
                    rbuf[0, i, pl.ds(j * 16, 16)] = z16
                return 0

            lax.fori_loop(0, CH, zfill, 0)

            def zcopy(k, _):
                pltpu.sync_copy(rbuf.at[0], acc.at[pl.ds(s * rpt + k * CH, CH)])
                return 0

            lax.fori_loop(0, rpt // CH, zcopy, 0)
            plsc.subcore_barrier()

            pltpu.async_copy(idx_hbm.at[wid, 0], ibuf.at[0], isem)

            def group_body(g, _):
                q = g % 2
                ib = ibuf.at[q]
                pltpu.make_async_copy(idx_hbm.at[wid, g], ib, isem).wait()

                @pl.when(g + 1 < groups)
                def _():
                    pltpu.async_copy(idx_hbm.at[wid, g + 1], ibuf.at[1 - q], isem)

                pltpu.async_copy(tbl.at[ib.at[0, 0]], rbuf.at[0], gsem0)
                for k in range(IG):
                    if k + 1 < IG:
                        pltpu.async_copy(
                            tbl.at[ib.at[0, k + 1]], rbuf.at[(k + 1) % 2],
                            gsems[(k + 1) % 2],
                        )
                    pltpu.make_async_copy(
                        tbl.at[ib.at[0, k]], rbuf.at[k % 2], gsems[k % 2]
                    ).wait()
                    pltpu.sync_copy(rbuf.at[k % 2], acc.at[ib.at[1, k]], add=True)
                return 0

            lax.fori_loop(0, groups, group_body, 0)
            plsc.subcore_barrier()
            pltpu.sync_copy(
                acc.at[pl.ds(s * rpt, rpt)],
                out_hbm.at[c, pl.ds(s * rpt, rpt), pl.ds(p * HD, HD)],
            )
            if p == 0:
                plsc.subcore_barrier()

    return agg_kernel


def _tc_first(x, w, degp, nacc):
    n = x.shape[0]
    grid = n // BN

    def body(x_ref, w_ref, dp_ref, dinv_ref, hs_ref):
        deg = 1.0 + dp_ref[0, :, 0] + dp_ref[1, :, 0]
        dinv = lax.rsqrt(deg)[:, None]
        m = jnp.dot(x_ref[...], w_ref[...], preferred_element_type=jnp.float32)
        dinv_ref[...] = jnp.broadcast_to(dinv, (BN, D))
        hs_ref[...] = dinv * m

    return pl.pallas_call(
        body,
        grid=(grid,),
        in_specs=[
            pl.BlockSpec((BN, D), lambda i: (i, 0)),
            pl.BlockSpec((D, D), lambda i: (0, 0)),
            pl.BlockSpec((NC, BN, D), lambda i: (0, i, 0)),
        ],
        out_specs=[
            pl.BlockSpec((BN, D), lambda i: (i, 0)),
            pl.BlockSpec((BN, D), lambda i: (i, 0)),
        ],
        out_shape=[
            jax.ShapeDtypeStruct((n, D), jnp.float32),
            jax.ShapeDtypeStruct((nacc, D), jnp.float32),
        ],
    )(x, w, degp)


def _tc_combine(sparts, hs, dinv_b, b, identity, w_next, relu, nacc):
    n = dinv_b.shape[0]
    grid = n // BN
    have_res = identity is not None
    have_mm = w_next is not None

    def body(*refs):
        i = 0
        sp_ref = refs[i]; i += 1
        hs_ref = refs[i]; i += 1
        dinv_ref = refs[i]; i += 1
        b_ref = refs[i]; i += 1
        id_ref = None
        w_ref = None
        if have_res:
            id_ref = refs[i]; i += 1
        if have_mm:
            w_ref = refs[i]; i += 1
        h_ref = refs[i]; i += 1
        hs2_ref = refs[i] if have_mm else None

        dinv = dinv_ref[...]
        agg = dinv * (sp_ref[0] + sp_ref[1] + hs_ref[...]) + b_ref[...]
        if have_res:
            agg = agg + id_ref[...]
        h = jnp.maximum(agg, 0.0) if relu else agg
        h_ref[...] = h
        if have_mm:
            m = jnp.dot(h, w_ref[...], preferred_element_type=jnp.float32)
            hs2_ref[...] = dinv * m

    in_specs = [
        pl.BlockSpec((NC, BN, D), lambda i: (0, i, 0)),
        pl.BlockSpec((BN, D), lambda i: (i, 0)),
        pl.BlockSpec((BN, D), lambda i: (i, 0)),
        pl.BlockSpec((1, D), lambda i: (0, 0)),
    ]
    args = [sparts, hs, dinv_b, b.reshape(1, D)]
    if have_res:
        in_specs.append(pl.BlockSpec((BN, D), lambda i: (i, 0)))
        args.append(identity)
    if have_mm:
        in_specs.append(pl.BlockSpec((D, D), lambda i: (0, 0)))
        args.append(w_next)

    out_specs = [pl.BlockSpec((BN, D), lambda i: (i, 0))]
    out_shape = [jax.ShapeDtypeStruct((n, D), jnp.float32)]
    if have_mm:
        out_specs.append(pl.BlockSpec((BN, D), lambda i: (i, 0)))
        out_shape.append(jax.ShapeDtypeStruct((nacc, D), jnp.float32))

    res = pl.pallas_call(
        body,
        grid=(grid,),
        in_specs=in_specs,
        out_specs=out_specs,
        out_shape=out_shape,
    )(*args)
    return res if have_mm else (res[0], None)


def kernel(x, edge_index, W0, b0, W1, b1, W2, b2, W3, b3):
    n, d = x.shape
    e = edge_index.shape[1]
    assert n == N and d == D and n % BN == 0

    nacc = -(-(n + 1) // (NS * CH)) * (NS * CH)
    groups = -(-e // (NW * IG * CH))
    ep = groups * IG * CH * NW

    src = jnp.concatenate([edge_index[0], jnp.zeros((ep - e,), jnp.int32)])
    dst = jnp.concatenate([edge_index[1], jnp.full((ep - e,), n, jnp.int32)])
    src_r = src.reshape(NW, groups, IG, CH)
    dst_r = dst.reshape(NW, groups, IG, CH)
    idx_pack = jnp.stack([src_r, dst_r], axis=2)

    degp = _sc_degree(nacc, groups)(idx_pack)
    dinv_b, hs = _tc_first(x, W0, degp, nacc)

    agg = _sc_aggregate(nacc, groups)
    Ws = [W1, W2, W3, None]
    bs = [b0, b1, b2, b3]
    h_prev = None
    for i in range(4):
        sparts = agg(hs, idx_pack)
        identity = h_prev if i in (1, 2) else None
        h, hs_next = _tc_combine(
            sparts, hs, dinv_b, bs[i], identity, Ws[i], relu=(i < 3), nacc=nacc
        )
        h_prev = h
        hs = hs_next
    return h_prev


# speedup vs baseline: 17.6319x; 1.0541x over previous
import functools

import jax
import jax.numpy as jnp
from jax import lax
from jax.experimental import pallas as pl
from jax.experimental.pallas import tpu as pltpu
from jax.experimental.pallas import tpu_sc as plsc

N = 10000
D = 128
NC = 2
NS = 16
NW = NC * NS
CH = 128
BN = 1000


def _mesh():
    return plsc.VectorSubcoreMesh(core_axis_name="c", subcore_axis_name="s")


IG = 16


DW = 16


def _sc_degree(nacc, groups):
    mesh = _mesh()
    rpt = nacc // NS

    @functools.partial(
        pl.kernel,
        mesh=mesh,
        out_type=jax.ShapeDtypeStruct((NC, nacc, DW), jnp.float32),
        compiler_params=pltpu.CompilerParams(use_tc_tiling_on_sc=False),
        scratch_types=[
            pltpu.VMEM((IG, CH), jnp.int32),
            pltpu.VMEM((CH, DW), jnp.float32),
            pltpu.VMEM_SHARED((nacc, DW), jnp.float32),
        ],
    )
    def deg_kernel(idx_hbm, out_hbm, dstv, ones, acc):
        c = lax.axis_index("c")
        s = lax.axis_index("s")
        wid = c * NS + s
        o16 = jnp.ones((16,), jnp.float32)
        z16 = jnp.zeros((16,), jnp.float32)

        def zfill(i, _):
            for j in range(DW // 16):
                ones[i, pl.ds(j * 16, 16)] = z16
            return 0

        def zcopy(k, _):
            pltpu.sync_copy(ones, acc.at[pl.ds(s * rpt + k * CH, CH)])
            return 0

        def fill(i, _):
            for j in range(DW // 16):
                ones[i, pl.ds(j * 16, 16)] = o16
            return 0

        lax.fori_loop(0, CH, zfill, 0)
        lax.fori_loop(0, rpt // CH, zcopy, 0)
        lax.fori_loop(0, CH, fill, 0)
        plsc.subcore_barrier()

        def body(g, _):
            pltpu.sync_copy(idx_hbm.at[wid, g, 1], dstv)
            for k in range(IG):
                pltpu.sync_copy(ones, acc.at[dstv.at[k]], add=True)
            return 0

        lax.fori_loop(0, groups, body, 0)
        plsc.subcore_barrier()
        pltpu.sync_copy(
            acc.at[pl.ds(s * rpt, rpt)], out_hbm.at[c, pl.ds(s * rpt, rpt)]
        )

    return deg_kernel


HD = D // 2


def _sc_aggregate(nacc, groups):
    mesh = _mesh()
    rpt = nacc // NS

    @functools.partial(
        pl.kernel,
        mesh=mesh,
        out_type=jax.ShapeDtypeStruct((NC, nacc, D), jnp.float32),
        compiler_params=pltpu.CompilerParams(use_tc_tiling_on_sc=False),
        scratch_types=[
            pltpu.VMEM((2, 2, IG, CH), jnp.int32),
            pltpu.VMEM((2, CH, HD), jnp.float32),
            pltpu.VMEM_SHARED((nacc, HD), jnp.float32),
            pltpu.VMEM_SHARED((nacc, HD), jnp.float32),
            pltpu.SemaphoreType.DMA,
            pltpu.SemaphoreType.DMA,
            pltpu.SemaphoreType.DMA,
        ],
    )
    def agg_kernel(hs_hbm, idx_hbm, out_hbm, ibuf, rbuf, tbl, acc, isem, gsem0, gsem1):
        c = lax.axis_index("c")
        s = lax.axis_index("s")
        wid = c * NS + s
        z16 = jnp.zeros((16,), jnp.float32)
        gsems = (gsem0, gsem1)

        for p in range(2):
            pltpu.sync_copy(
                hs_hbm.at[pl.ds(s * rpt, rpt), pl.ds(p * HD, HD)],
                tbl.at[pl.ds(s * rpt, rpt)],
            )

            def zfill(i, _):
                for j in range(HD // 16):
                    rbuf[0, i, pl.ds(j * 16, 16)] = z16
                return 0

            lax.fori_loop(0, CH, zfill, 0)

            def zcopy(k, _):
                pltpu.sync_copy(rbuf.at[0], acc.at[pl.ds(s * rpt + k * CH, CH)])
                return 0

            lax.fori_loop(0, rpt // CH, zcopy, 0)
            plsc.subcore_barrier()

            pltpu.async_copy(idx_hbm.at[wid, 0], ibuf.at[0], isem)

            def group_body(g, _):
                q = g % 2
                ib = ibuf.at[q]
                pltpu.make_async_copy(idx_hbm.at[wid, g], ib, isem).wait()

                @pl.when(g + 1 < groups)
                def _():
                    pltpu.async_copy(idx_hbm.at[wid, g + 1], ibuf.at[1 - q], isem)

                pltpu.async_copy(tbl.at[ib.at[0, 0]], rbuf.at[0], gsem0)
                for k in range(IG):
                    if k + 1 < IG:
                        pltpu.async_copy(
                            tbl.at[ib.at[0, k + 1]], rbuf.at[(k + 1) % 2],
                            gsems[(k + 1) % 2],
                        )
                    pltpu.make_async_copy(
                        tbl.at[ib.at[0, k]], rbuf.at[k % 2], gsems[k % 2]
                    ).wait()
                    pltpu.sync_copy(rbuf.at[k % 2], acc.at[ib.at[1, k]], add=True)
                return 0

            lax.fori_loop(0, groups, group_body, 0)
            plsc.subcore_barrier()
            pltpu.sync_copy(
                acc.at[pl.ds(s * rpt, rpt)],
                out_hbm.at[c, pl.ds(s * rpt, rpt), pl.ds(p * HD, HD)],
            )
            if p == 0:
                plsc.subcore_barrier()

    return agg_kernel


def _tc_first(x, w, degp, nacc):
    n = x.shape[0]
    grid = n // BN

    def body(x_ref, w_ref, dp_ref, dinv_ref, hs_ref):
        deg = 1.0 + dp_ref[0, :, 0] + dp_ref[1, :, 0]
        dinv = lax.rsqrt(deg)[:, None]
        m = jnp.dot(x_ref[...], w_ref[...], preferred_element_type=jnp.float32)
        dinv_ref[...] = jnp.broadcast_to(dinv, (BN, D))
        hs_ref[...] = dinv * m

    return pl.pallas_call(
        body,
        grid=(grid,),
        in_specs=[
            pl.BlockSpec((BN, D), lambda i: (i, 0)),
            pl.BlockSpec((D, D), lambda i: (0, 0)),
            pl.BlockSpec((NC, BN, DW), lambda i: (0, i, 0)),
        ],
        out_specs=[
            pl.BlockSpec((BN, D), lambda i: (i, 0)),
            pl.BlockSpec((BN, D), lambda i: (i, 0)),
        ],
        out_shape=[
            jax.ShapeDtypeStruct((n, D), jnp.float32),
            jax.ShapeDtypeStruct((nacc, D), jnp.float32),
        ],
    )(x, w, degp)


def _tc_combine(sparts, hs, dinv_b, b, identity, w_next, relu, nacc):
    n = dinv_b.shape[0]
    grid = n // BN
    have_res = identity is not None
    have_mm = w_next is not None

    def body(*refs):
        i = 0
        sp_ref = refs[i]; i += 1
        hs_ref = refs[i]; i += 1
        dinv_ref = refs[i]; i += 1
        b_ref = refs[i]; i += 1
        id_ref = None
        w_ref = None
        if have_res:
            id_ref = refs[i]; i += 1
        if have_mm:
            w_ref = refs[i]; i += 1
        h_ref = refs[i]; i += 1
        hs2_ref = refs[i] if have_mm else None

        dinv = dinv_ref[...]
        agg = dinv * (sp_ref[0] + sp_ref[1] + hs_ref[...]) + b_ref[...]
        if have_res:
            agg = agg + id_ref[...]
        h = jnp.maximum(agg, 0.0) if relu else agg
        h_ref[...] = h
        if have_mm:
            m = jnp.dot(h, w_ref[...], preferred_element_type=jnp.float32)
            hs2_ref[...] = dinv * m

    in_specs = [
        pl.BlockSpec((NC, BN, D), lambda i: (0, i, 0)),
        pl.BlockSpec((BN, D), lambda i: (i, 0)),
        pl.BlockSpec((BN, D), lambda i: (i, 0)),
        pl.BlockSpec((1, D), lambda i: (0, 0)),
    ]
    args = [sparts, hs, dinv_b, b.reshape(1, D)]
    if have_res:
        in_specs.append(pl.BlockSpec((BN, D), lambda i: (i, 0)))
        args.append(identity)
    if have_mm:
        in_specs.append(pl.BlockSpec((D, D), lambda i: (0, 0)))
        args.append(w_next)

    out_specs = [pl.BlockSpec((BN, D), lambda i: (i, 0))]
    out_shape = [jax.ShapeDtypeStruct((n, D), jnp.float32)]
    if have_mm:
        out_specs.append(pl.BlockSpec((BN, D), lambda i: (i, 0)))
        out_shape.append(jax.ShapeDtypeStruct((nacc, D), jnp.float32))

    res = pl.pallas_call(
        body,
        grid=(grid,),
        in_specs=in_specs,
        out_specs=out_specs,
        out_shape=out_shape,
    )(*args)
    return res if have_mm else (res[0], None)


def kernel(x, edge_index, W0, b0, W1, b1, W2, b2, W3, b3):
    n, d = x.shape
    e = edge_index.shape[1]
    assert n == N and d == D and n % BN == 0

    nacc = -(-(n + 1) // (NS * CH)) * (NS * CH)
    groups = -(-e // (NW * IG * CH))
    ep = groups * IG * CH * NW

    src = jnp.concatenate([edge_index[0], jnp.zeros((ep - e,), jnp.int32)])
    dst = jnp.concatenate([edge_index[1], jnp.full((ep - e,), n, jnp.int32)])
    src_r = src.reshape(NW, groups, IG, CH)
    dst_r = dst.reshape(NW, groups, IG, CH)
    idx_pack = jnp.stack([src_r, dst_r], axis=2)

    degp = _sc_degree(nacc, groups)(idx_pack)
    dinv_b, hs = _tc_first(x, W0, degp, nacc)

    agg = _sc_aggregate(nacc, groups)
    Ws = [W1, W2, W3, None]
    bs = [b0, b1, b2, b3]
    h_prev = None
    for i in range(4):
        sparts = agg(hs, idx_pack)
        identity = h_prev if i in (1, 2) else None
        h, hs_next = _tc_combine(
            sparts, hs, dinv_b, bs[i], identity, Ws[i], relu=(i < 3), nacc=nacc
        )
        h_prev = h
        hs = hs_next
    return h_prev


# speedup vs baseline: 20.3174x; 1.1523x over previous
import functools

import jax
import jax.numpy as jnp
from jax import lax
from jax.experimental import pallas as pl
from jax.experimental.pallas import tpu as pltpu
from jax.experimental.pallas import tpu_sc as plsc

N = 10000
D = 128
NC = 2
NS = 16
NW = NC * NS
CH = 128
BN = 1000


def _mesh():
    return plsc.VectorSubcoreMesh(core_axis_name="c", subcore_axis_name="s")


IG = 16


DW = 16


def _sc_degree(nacc, groups):
    mesh = _mesh()
    rpt = nacc // NS

    @functools.partial(
        pl.kernel,
        mesh=mesh,
        out_type=jax.ShapeDtypeStruct((NC, nacc, DW), jnp.float32),
        compiler_params=pltpu.CompilerParams(use_tc_tiling_on_sc=False),
        scratch_types=[
            pltpu.VMEM((IG, CH), jnp.int32),
            pltpu.VMEM((CH, DW), jnp.float32),
            pltpu.VMEM_SHARED((nacc, DW), jnp.float32),
        ],
    )
    def deg_kernel(idx_hbm, out_hbm, dstv, ones, acc):
        c = lax.axis_index("c")
        s = lax.axis_index("s")
        wid = c * NS + s
        o16 = jnp.ones((16,), jnp.float32)
        z16 = jnp.zeros((16,), jnp.float32)

        def zfill(i, _):
            for j in range(DW // 16):
                ones[i, pl.ds(j * 16, 16)] = z16
            return 0

        def zcopy(k, _):
            pltpu.sync_copy(ones, acc.at[pl.ds(s * rpt + k * CH, CH)])
            return 0

        def fill(i, _):
            for j in range(DW // 16):
                ones[i, pl.ds(j * 16, 16)] = o16
            return 0

        lax.fori_loop(0, CH, zfill, 0)
        lax.fori_loop(0, rpt // CH, zcopy, 0)
        lax.fori_loop(0, CH, fill, 0)
        plsc.subcore_barrier()

        def body(g, _):
            pltpu.sync_copy(idx_hbm.at[wid, g, 1], dstv)
            for k in range(IG):
                pltpu.sync_copy(ones, acc.at[dstv.at[k]], add=True)
            return 0

        lax.fori_loop(0, groups, body, 0)
        plsc.subcore_barrier()
        pltpu.sync_copy(
            acc.at[pl.ds(s * rpt, rpt)], out_hbm.at[c, pl.ds(s * rpt, rpt)]
        )

    return deg_kernel


HD = D // 2


def _sc_aggregate(nacc, groups):
    mesh = _mesh()
    rpt = nacc // NS

    @functools.partial(
        pl.kernel,
        mesh=mesh,
        out_type=jax.ShapeDtypeStruct((NC, nacc, D), jnp.float32),
        compiler_params=pltpu.CompilerParams(use_tc_tiling_on_sc=False),
        scratch_types=[
            pltpu.VMEM((2, 2, IG, CH), jnp.int32),
            pltpu.VMEM((4, CH, HD), jnp.float32),
            pltpu.VMEM_SHARED((nacc, HD), jnp.float32),
            pltpu.VMEM_SHARED((nacc, HD), jnp.float32),
            pltpu.SemaphoreType.DMA,
            ((pltpu.SemaphoreType.DMA,) * 4),
            ((pltpu.SemaphoreType.DMA,) * 4),
        ],
    )
    def agg_kernel(hs_hbm, idx_hbm, out_hbm, ibuf, rbuf, tbl, acc, isem, gsems, ssems):
        c = lax.axis_index("c")
        s = lax.axis_index("s")
        wid = c * NS + s
        z16 = jnp.zeros((16,), jnp.float32)

        for p in range(2):
            pltpu.sync_copy(
                hs_hbm.at[pl.ds(s * rpt, rpt), pl.ds(p * HD, HD)],
                tbl.at[pl.ds(s * rpt, rpt)],
            )

            def zfill(i, _):
                for j in range(HD // 16):
                    rbuf[0, i, pl.ds(j * 16, 16)] = z16
                return 0

            lax.fori_loop(0, CH, zfill, 0)

            def zcopy(k, _):
                pltpu.sync_copy(rbuf.at[0], acc.at[pl.ds(s * rpt + k * CH, CH)])
                return 0

            lax.fori_loop(0, rpt // CH, zcopy, 0)
            plsc.subcore_barrier()

            pltpu.async_copy(idx_hbm.at[wid, 0], ibuf.at[0], isem)

            def group_body(g, _):
                q = g % 2
                ib = ibuf.at[q]
                pltpu.make_async_copy(idx_hbm.at[wid, g], ib, isem).wait()

                @pl.when(g + 1 < groups)
                def _():
                    pltpu.async_copy(idx_hbm.at[wid, g + 1], ibuf.at[1 - q], isem)

                pltpu.async_copy(tbl.at[ib.at[0, 0]], rbuf.at[0], gsems[0])
                for k in range(IG):
                    j = k % 4
                    if k >= 3:
                        pltpu.make_async_copy(
                            rbuf.at[(k - 3) % 4], acc.at[ib.at[1, k - 3]],
                            ssems[(k - 3) % 4],
                        ).wait()
                    if k + 1 < IG:
                        pltpu.async_copy(
                            tbl.at[ib.at[0, k + 1]], rbuf.at[(k + 1) % 4],
                            gsems[(k + 1) % 4],
                        )
                    pltpu.make_async_copy(
                        tbl.at[ib.at[0, k]], rbuf.at[j], gsems[j]
                    ).wait()
                    pltpu.async_copy(rbuf.at[j], acc.at[ib.at[1, k]], ssems[j], add=True)
                for k in range(IG - 3, IG):
                    pltpu.make_async_copy(
                        rbuf.at[k % 4], acc.at[ib.at[1, k]], ssems[k % 4]
                    ).wait()
                return 0

            lax.fori_loop(0, groups, group_body, 0)
            plsc.subcore_barrier()
            pltpu.sync_copy(
                acc.at[pl.ds(s * rpt, rpt)],
                out_hbm.at[c, pl.ds(s * rpt, rpt), pl.ds(p * HD, HD)],
            )
            if p == 0:
                plsc.subcore_barrier()

    return agg_kernel


def _tc_first(x, w, degp, nacc):
    n = x.shape[0]
    grid = n // BN

    def body(x_ref, w_ref, dp_ref, dinv_ref, hs_ref):
        deg = 1.0 + dp_ref[0, :, 0] + dp_ref[1, :, 0]
        dinv = lax.rsqrt(deg)[:, None]
        m = jnp.dot(x_ref[...], w_ref[...], preferred_element_type=jnp.float32)
        dinv_ref[...] = jnp.broadcast_to(dinv, (BN, D))
        hs_ref[...] = dinv * m

    return pl.pallas_call(
        body,
        grid=(grid,),
        in_specs=[
            pl.BlockSpec((BN, D), lambda i: (i, 0)),
            pl.BlockSpec((D, D), lambda i: (0, 0)),
            pl.BlockSpec((NC, BN, DW), lambda i: (0, i, 0)),
        ],
        out_specs=[
            pl.BlockSpec((BN, D), lambda i: (i, 0)),
            pl.BlockSpec((BN, D), lambda i: (i, 0)),
        ],
        out_shape=[
            jax.ShapeDtypeStruct((n, D), jnp.float32),
            jax.ShapeDtypeStruct((nacc, D), jnp.float32),
        ],
    )(x, w, degp)


def _tc_combine(sparts, hs, dinv_b, b, identity, w_next, relu, nacc):
    n = dinv_b.shape[0]
    grid = n // BN
    have_res = identity is not None
    have_mm = w_next is not None

    def body(*refs):
        i = 0
        sp_ref = refs[i]; i += 1
        hs_ref = refs[i]; i += 1
        dinv_ref = refs[i]; i += 1
        b_ref = refs[i]; i += 1
        id_ref = None
        w_ref = None
        if have_res:
            id_ref = refs[i]; i += 1
        if have_mm:
            w_ref = refs[i]; i += 1
        h_ref = refs[i]; i += 1
        hs2_ref = refs[i] if have_mm else None

        dinv = dinv_ref[...]
        agg = dinv * (sp_ref[0] + sp_ref[1] + hs_ref[...]) + b_ref[...]
        if have_res:
            agg = agg + id_ref[...]
        h = jnp.maximum(agg, 0.0) if relu else agg
        h_ref[...] = h
        if have_mm:
            m = jnp.dot(h, w_ref[...], preferred_element_type=jnp.float32)
            hs2_ref[...] = dinv * m

    in_specs = [
        pl.BlockSpec((NC, BN, D), lambda i: (0, i, 0)),
        pl.BlockSpec((BN, D), lambda i: (i, 0)),
        pl.BlockSpec((BN, D), lambda i: (i, 0)),
        pl.BlockSpec((1, D), lambda i: (0, 0)),
    ]
    args = [sparts, hs, dinv_b, b.reshape(1, D)]
    if have_res:
        in_specs.append(pl.BlockSpec((BN, D), lambda i: (i, 0)))
        args.append(identity)
    if have_mm:
        in_specs.append(pl.BlockSpec((D, D), lambda i: (0, 0)))
        args.append(w_next)

    out_specs = [pl.BlockSpec((BN, D), lambda i: (i, 0))]
    out_shape = [jax.ShapeDtypeStruct((n, D), jnp.float32)]
    if have_mm:
        out_specs.append(pl.BlockSpec((BN, D), lambda i: (i, 0)))
        out_shape.append(jax.ShapeDtypeStruct((nacc, D), jnp.float32))

    res = pl.pallas_call(
        body,
        grid=(grid,),
        in_specs=in_specs,
        out_specs=out_specs,
        out_shape=out_shape,
    )(*args)
    return res if have_mm else (res[0], None)


def kernel(x, edge_index, W0, b0, W1, b1, W2, b2, W3, b3):
    n, d = x.shape
    e = edge_index.shape[1]
    assert n == N and d == D and n % BN == 0

    nacc = -(-(n + 1) // (NS * CH)) * (NS * CH)
    groups = -(-e // (NW * IG * CH))
    ep = groups * IG * CH * NW

    src = jnp.concatenate([edge_index[0], jnp.zeros((ep - e,), jnp.int32)])
    dst = jnp.concatenate([edge_index[1], jnp.full((ep - e,), n, jnp.int32)])
    src_r = src.reshape(NW, groups, IG, CH)
    dst_r = dst.reshape(NW, groups, IG, CH)
    idx_pack = jnp.stack([src_r, dst_r], axis=2)

    degp = _sc_degree(nacc, groups)(idx_pack)
    dinv_b, hs = _tc_first(x, W0, degp, nacc)

    agg = _sc_aggregate(nacc, groups)
    Ws = [W1, W2, W3, None]
    bs = [b0, b1, b2, b3]
    h_prev = None
    for i in range(4):
        sparts = agg(hs, idx_pack)
        identity = h_prev if i in (1, 2) else None
        h, hs_next = _tc_combine(
            sparts, hs, dinv_b, bs[i], identity, Ws[i], relu=(i < 3), nacc=nacc
        )
        h_prev = h
        hs = hs_next
    return h_prev


# speedup vs baseline: 20.3293x; 1.0006x over previous
import functools

import jax
import jax.numpy as jnp
from jax import lax
from jax.experimental import pallas as pl
from jax.experimental.pallas import tpu as pltpu
from jax.experimental.pallas import tpu_sc as plsc

N = 10000
D = 128
NC = 2
NS = 16
NW = NC * NS
CH = 128
BN = 1000


def _mesh():
    return plsc.VectorSubcoreMesh(core_axis_name="c", subcore_axis_name="s")


IG = 16


DW = 16


def _sc_degree(nacc, groups):
    mesh = _mesh()
    rpt = nacc // NS

    @functools.partial(
        pl.kernel,
        mesh=mesh,
        out_type=jax.ShapeDtypeStruct((NC, nacc, DW), jnp.float32),
        compiler_params=pltpu.CompilerParams(use_tc_tiling_on_sc=False),
        scratch_types=[
            pltpu.VMEM((IG, CH), jnp.int32),
            pltpu.VMEM((CH, DW), jnp.float32),
            pltpu.VMEM_SHARED((nacc, DW), jnp.float32),
        ],
    )
    def deg_kernel(dst_hbm, out_hbm, dstv, ones, acc):
        c = lax.axis_index("c")
        s = lax.axis_index("s")
        wid = c * NS + s
        o16 = jnp.ones((16,), jnp.float32)
        z16 = jnp.zeros((16,), jnp.float32)

        def zfill(i, _):
            for j in range(DW // 16):
                ones[i, pl.ds(j * 16, 16)] = z16
            return 0

        def zcopy(k, _):
            pltpu.sync_copy(ones, acc.at[pl.ds(s * rpt + k * CH, CH)])
            return 0

        def fill(i, _):
            for j in range(DW // 16):
                ones[i, pl.ds(j * 16, 16)] = o16
            return 0

        lax.fori_loop(0, CH, zfill, 0)
        lax.fori_loop(0, rpt // CH, zcopy, 0)
        lax.fori_loop(0, CH, fill, 0)
        plsc.subcore_barrier()

        def body(g, _):
            pltpu.sync_copy(dst_hbm.at[wid, g], dstv)
            for k in range(IG):
                pltpu.sync_copy(ones, acc.at[dstv.at[k]], add=True)
            return 0

        lax.fori_loop(0, groups, body, 0)
        plsc.subcore_barrier()
        pltpu.sync_copy(
            acc.at[pl.ds(s * rpt, rpt)], out_hbm.at[c, pl.ds(s * rpt, rpt)]
        )

    return deg_kernel


HD = D // 2


def _sc_aggregate(nacc, groups):
    mesh = _mesh()
    rpt = nacc // NS

    @functools.partial(
        pl.kernel,
        mesh=mesh,
        out_type=jax.ShapeDtypeStruct((NC, nacc, D), jnp.float32),
        compiler_params=pltpu.CompilerParams(use_tc_tiling_on_sc=False),
        scratch_types=[
            pltpu.VMEM((2, 2, IG, CH), jnp.int32),
            pltpu.VMEM((4, CH, HD), jnp.float32),
            pltpu.VMEM_SHARED((nacc, HD), jnp.float32),
            pltpu.VMEM_SHARED((nacc, HD), jnp.float32),
            pltpu.SemaphoreType.DMA,
            ((pltpu.SemaphoreType.DMA,) * 4),
            ((pltpu.SemaphoreType.DMA,) * 4),
        ],
    )
    def agg_kernel(hs_hbm, src_hbm, dst_hbm, out_hbm, ibuf, rbuf, tbl, acc, isem, gsems, ssems):
        c = lax.axis_index("c")
        s = lax.axis_index("s")
        wid = c * NS + s
        z16 = jnp.zeros((16,), jnp.float32)

        for p in range(2):
            pltpu.sync_copy(
                hs_hbm.at[pl.ds(s * rpt, rpt), pl.ds(p * HD, HD)],
                tbl.at[pl.ds(s * rpt, rpt)],
            )

            def zfill(i, _):
                for j in range(HD // 16):
                    rbuf[0, i, pl.ds(j * 16, 16)] = z16
                return 0

            lax.fori_loop(0, CH, zfill, 0)

            def zcopy(k, _):
                pltpu.sync_copy(rbuf.at[0], acc.at[pl.ds(s * rpt + k * CH, CH)])
                return 0

            lax.fori_loop(0, rpt // CH, zcopy, 0)
            plsc.subcore_barrier()

            pltpu.async_copy(src_hbm.at[wid, 0], ibuf.at[0, 0], isem)
            pltpu.async_copy(dst_hbm.at[wid, 0], ibuf.at[0, 1], isem)

            def group_body(g, _):
                q = g % 2
                ibs = ibuf.at[q, 0]
                ibd = ibuf.at[q, 1]
                pltpu.make_async_copy(src_hbm.at[wid, g], ibs, isem).wait()
                pltpu.make_async_copy(dst_hbm.at[wid, g], ibd, isem).wait()

                @pl.when(g + 1 < groups)
                def _():
                    pltpu.async_copy(src_hbm.at[wid, g + 1], ibuf.at[1 - q, 0], isem)
                    pltpu.async_copy(dst_hbm.at[wid, g + 1], ibuf.at[1 - q, 1], isem)

                pltpu.async_copy(tbl.at[ibs.at[0]], rbuf.at[0], gsems[0])
                for k in range(IG):
                    j = k % 4
                    if k >= 3:
                        pltpu.make_async_copy(
                            rbuf.at[(k - 3) % 4], acc.at[ibd.at[k - 3]],
                            ssems[(k - 3) % 4],
                        ).wait()
                    if k + 1 < IG:
                        pltpu.async_copy(
                            tbl.at[ibs.at[k + 1]], rbuf.at[(k + 1) % 4],
                            gsems[(k + 1) % 4],
                        )
                    pltpu.make_async_copy(
                        tbl.at[ibs.at[k]], rbuf.at[j], gsems[j]
                    ).wait()
                    pltpu.async_copy(rbuf.at[j], acc.at[ibd.at[k]], ssems[j], add=True)
                for k in range(IG - 3, IG):
                    pltpu.make_async_copy(
                        rbuf.at[k % 4], acc.at[ibd.at[k]], ssems[k % 4]
                    ).wait()
                return 0

            lax.fori_loop(0, groups, group_body, 0)
            plsc.subcore_barrier()
            pltpu.sync_copy(
                acc.at[pl.ds(s * rpt, rpt)],
                out_hbm.at[c, pl.ds(s * rpt, rpt), pl.ds(p * HD, HD)],
            )
            if p == 0:
                plsc.subcore_barrier()

    return agg_kernel


def _tc_first(x, w, degp, nacc):
    n = x.shape[0]
    grid = n // BN

    def body(x_ref, w_ref, dp_ref, dinv_ref, hs_ref):
        deg = 1.0 + dp_ref[0, :, 0] + dp_ref[1, :, 0]
        dinv = lax.rsqrt(deg)[:, None]
        m = jnp.dot(x_ref[...], w_ref[...], preferred_element_type=jnp.float32)
        dinv_ref[...] = jnp.broadcast_to(dinv, (BN, D))
        hs_ref[...] = dinv * m

    return pl.pallas_call(
        body,
        grid=(grid,),
        in_specs=[
            pl.BlockSpec((BN, D), lambda i: (i, 0)),
            pl.BlockSpec((D, D), lambda i: (0, 0)),
            pl.BlockSpec((NC, BN, DW), lambda i: (0, i, 0)),
        ],
        out_specs=[
            pl.BlockSpec((BN, D), lambda i: (i, 0)),
            pl.BlockSpec((BN, D), lambda i: (i, 0)),
        ],
        out_shape=[
            jax.ShapeDtypeStruct((n, D), jnp.float32),
            jax.ShapeDtypeStruct((nacc, D), jnp.float32),
        ],
    )(x, w, degp)


def _tc_combine(sparts, hs, dinv_b, b, identity, w_next, relu, nacc):
    n = dinv_b.shape[0]
    grid = n // BN
    have_res = identity is not None
    have_mm = w_next is not None

    def body(*refs):
        i = 0
        sp_ref = refs[i]; i += 1
        hs_ref = refs[i]; i += 1
        dinv_ref = refs[i]; i += 1
        b_ref = refs[i]; i += 1
        id_ref = None
        w_ref = None
        if have_res:
            id_ref = refs[i]; i += 1
        if have_mm:
            w_ref = refs[i]; i += 1
        h_ref = refs[i]; i += 1
        hs2_ref = refs[i] if have_mm else None

        dinv = dinv_ref[...]
        agg = dinv * (sp_ref[0] + sp_ref[1] + hs_ref[...]) + b_ref[...]
        if have_res:
            agg = agg + id_ref[...]
        h = jnp.maximum(agg, 0.0) if relu else agg
        h_ref[...] = h
        if have_mm:
            m = jnp.dot(h, w_ref[...], preferred_element_type=jnp.float32)
            hs2_ref[...] = dinv * m

    in_specs = [
        pl.BlockSpec((NC, BN, D), lambda i: (0, i, 0)),
        pl.BlockSpec((BN, D), lambda i: (i, 0)),
        pl.BlockSpec((BN, D), lambda i: (i, 0)),
        pl.BlockSpec((1, D), lambda i: (0, 0)),
    ]
    args = [sparts, hs, dinv_b, b.reshape(1, D)]
    if have_res:
        in_specs.append(pl.BlockSpec((BN, D), lambda i: (i, 0)))
        args.append(identity)
    if have_mm:
        in_specs.append(pl.BlockSpec((D, D), lambda i: (0, 0)))
        args.append(w_next)

    out_specs = [pl.BlockSpec((BN, D), lambda i: (i, 0))]
    out_shape = [jax.ShapeDtypeStruct((n, D), jnp.float32)]
    if have_mm:
        out_specs.append(pl.BlockSpec((BN, D), lambda i: (i, 0)))
        out_shape.append(jax.ShapeDtypeStruct((nacc, D), jnp.float32))

    res = pl.pallas_call(
        body,
        grid=(grid,),
        in_specs=in_specs,
        out_specs=out_specs,
        out_shape=out_shape,
    )(*args)
    return res if have_mm else (res[0], None)


def kernel(x, edge_index, W0, b0, W1, b1, W2, b2, W3, b3):
    n, d = x.shape
    e = edge_index.shape[1]
    assert n == N and d == D and n % BN == 0

    nacc = -(-(n + 1) // (NS * CH)) * (NS * CH)
    groups = -(-e // (NW * IG * CH))
    ep = groups * IG * CH * NW

    src_r = jnp.pad(edge_index[0], (0, ep - e)).reshape(NW, groups, IG, CH)
    dst_r = jnp.pad(edge_index[1], (0, ep - e), constant_values=n).reshape(
        NW, groups, IG, CH
    )

    degp = _sc_degree(nacc, groups)(dst_r)
    dinv_b, hs = _tc_first(x, W0, degp, nacc)

    agg = _sc_aggregate(nacc, groups)
    Ws = [W1, W2, W3, None]
    bs = [b0, b1, b2, b3]
    h_prev = None
    for i in range(4):
        sparts = agg(hs, src_r, dst_r)
        identity = h_prev if i in (1, 2) else None
        h, hs_next = _tc_combine(
            sparts, hs, dinv_b, bs[i], identity, Ws[i], relu=(i < 3), nacc=nacc
        )
        h_prev = h
        hs = hs_next
    return h_prev


# speedup vs baseline: 20.7883x; 1.0226x over previous
import functools

import jax
import jax.numpy as jnp
from jax import lax
from jax.experimental import pallas as pl
from jax.experimental.pallas import tpu as pltpu
from jax.experimental.pallas import tpu_sc as plsc

N = 10000
D = 128
NC = 2
NS = 16
NW = NC * NS
CH = 128
BN = 1000


def _mesh():
    return plsc.VectorSubcoreMesh(core_axis_name="c", subcore_axis_name="s")


IG = 20


DW = 16


def _sc_degree(nacc, groups):
    mesh = _mesh()
    rpt = nacc // NS

    @functools.partial(
        pl.kernel,
        mesh=mesh,
        out_type=jax.ShapeDtypeStruct((NC, nacc, DW), jnp.float32),
        compiler_params=pltpu.CompilerParams(use_tc_tiling_on_sc=False),
        scratch_types=[
            pltpu.VMEM((IG, CH), jnp.int32),
            pltpu.VMEM((CH, DW), jnp.float32),
            pltpu.VMEM_SHARED((nacc, DW), jnp.float32),
        ],
    )
    def deg_kernel(dst_hbm, out_hbm, dstv, ones, acc):
        c = lax.axis_index("c")
        s = lax.axis_index("s")
        wid = c * NS + s
        o16 = jnp.ones((16,), jnp.float32)
        z16 = jnp.zeros((16,), jnp.float32)

        def zfill(i, _):
            for j in range(DW // 16):
                ones[i, pl.ds(j * 16, 16)] = z16
            return 0

        def zcopy(k, _):
            pltpu.sync_copy(ones, acc.at[pl.ds(s * rpt + k * CH, CH)])
            return 0

        def fill(i, _):
            for j in range(DW // 16):
                ones[i, pl.ds(j * 16, 16)] = o16
            return 0

        lax.fori_loop(0, CH, zfill, 0)
        lax.fori_loop(0, rpt // CH, zcopy, 0)
        lax.fori_loop(0, CH, fill, 0)
        plsc.subcore_barrier()

        def body(g, _):
            pltpu.sync_copy(dst_hbm.at[wid, g], dstv)
            for k in range(IG):
                pltpu.sync_copy(ones, acc.at[dstv.at[k]], add=True)
            return 0

        lax.fori_loop(0, groups, body, 0)
        plsc.subcore_barrier()
        pltpu.sync_copy(
            acc.at[pl.ds(s * rpt, rpt)], out_hbm.at[c, pl.ds(s * rpt, rpt)]
        )

    return deg_kernel


HD = D // 2


def _sc_aggregate(nacc, groups):
    mesh = _mesh()
    rpt = nacc // NS

    @functools.partial(
        pl.kernel,
        mesh=mesh,
        out_type=jax.ShapeDtypeStruct((NC, nacc, D), jnp.float32),
        compiler_params=pltpu.CompilerParams(use_tc_tiling_on_sc=False),
        scratch_types=[
            pltpu.VMEM((2, 2, IG, CH), jnp.int32),
            pltpu.VMEM((4, CH, HD), jnp.float32),
            pltpu.VMEM_SHARED((nacc, HD), jnp.float32),
            pltpu.VMEM_SHARED((nacc, HD), jnp.float32),
            pltpu.SemaphoreType.DMA,
            ((pltpu.SemaphoreType.DMA,) * 4),
            ((pltpu.SemaphoreType.DMA,) * 4),
        ],
    )
    def agg_kernel(hs_hbm, src_hbm, dst_hbm, out_hbm, ibuf, rbuf, tbl, acc, isem, gsems, ssems):
        c = lax.axis_index("c")
        s = lax.axis_index("s")
        wid = c * NS + s
        z16 = jnp.zeros((16,), jnp.float32)

        for p in range(2):
            pltpu.sync_copy(
                hs_hbm.at[pl.ds(s * rpt, rpt), pl.ds(p * HD, HD)],
                tbl.at[pl.ds(s * rpt, rpt)],
            )

            def zfill(i, _):
                for j in range(HD // 16):
                    rbuf[0, i, pl.ds(j * 16, 16)] = z16
                return 0

            lax.fori_loop(0, CH, zfill, 0)

            def zcopy(k, _):
                pltpu.sync_copy(rbuf.at[0], acc.at[pl.ds(s * rpt + k * CH, CH)])
                return 0

            lax.fori_loop(0, rpt // CH, zcopy, 0)
            plsc.subcore_barrier()

            pltpu.async_copy(src_hbm.at[wid, 0], ibuf.at[0, 0], isem)
            pltpu.async_copy(dst_hbm.at[wid, 0], ibuf.at[0, 1], isem)

            def group_body(g, _):
                q = g % 2
                ibs = ibuf.at[q, 0]
                ibd = ibuf.at[q, 1]
                pltpu.make_async_copy(src_hbm.at[wid, g], ibs, isem).wait()
                pltpu.make_async_copy(dst_hbm.at[wid, g], ibd, isem).wait()

                @pl.when(g + 1 < groups)
                def _():
                    pltpu.async_copy(src_hbm.at[wid, g + 1], ibuf.at[1 - q, 0], isem)
                    pltpu.async_copy(dst_hbm.at[wid, g + 1], ibuf.at[1 - q, 1], isem)

                pltpu.async_copy(tbl.at[ibs.at[0]], rbuf.at[0], gsems[0])
                for k in range(IG):
                    j = k % 4
                    if k >= 3:
                        pltpu.make_async_copy(
                            rbuf.at[(k - 3) % 4], acc.at[ibd.at[k - 3]],
                            ssems[(k - 3) % 4],
                        ).wait()
                    if k + 1 < IG:
                        pltpu.async_copy(
                            tbl.at[ibs.at[k + 1]], rbuf.at[(k + 1) % 4],
                            gsems[(k + 1) % 4],
                        )
                    pltpu.make_async_copy(
                        tbl.at[ibs.at[k]], rbuf.at[j], gsems[j]
                    ).wait()
                    pltpu.async_copy(rbuf.at[j], acc.at[ibd.at[k]], ssems[j], add=True)
                for k in range(IG - 3, IG):
                    pltpu.make_async_copy(
                        rbuf.at[k % 4], acc.at[ibd.at[k]], ssems[k % 4]
                    ).wait()
                return 0

            lax.fori_loop(0, groups, group_body, 0)
            plsc.subcore_barrier()
            pltpu.sync_copy(
                acc.at[pl.ds(s * rpt, rpt)],
                out_hbm.at[c, pl.ds(s * rpt, rpt), pl.ds(p * HD, HD)],
            )
            if p == 0:
                plsc.subcore_barrier()

    return agg_kernel


def _tc_first(x, w, degp, nacc):
    n = x.shape[0]
    grid = n // BN

    def body(x_ref, w_ref, dp_ref, dinv_ref, hs_ref):
        deg = 1.0 + dp_ref[0, :, 0] + dp_ref[1, :, 0]
        dinv = lax.rsqrt(deg)[:, None]
        m = jnp.dot(x_ref[...], w_ref[...], preferred_element_type=jnp.float32)
        dinv_ref[...] = jnp.broadcast_to(dinv, (BN, D))
        hs_ref[...] = dinv * m

    return pl.pallas_call(
        body,
        grid=(grid,),
        in_specs=[
            pl.BlockSpec((BN, D), lambda i: (i, 0)),
            pl.BlockSpec((D, D), lambda i: (0, 0)),
            pl.BlockSpec((NC, BN, DW), lambda i: (0, i, 0)),
        ],
        out_specs=[
            pl.BlockSpec((BN, D), lambda i: (i, 0)),
            pl.BlockSpec((BN, D), lambda i: (i, 0)),
        ],
        out_shape=[
            jax.ShapeDtypeStruct((n, D), jnp.float32),
            jax.ShapeDtypeStruct((nacc, D), jnp.float32),
        ],
    )(x, w, degp)


def _tc_combine(sparts, hs, dinv_b, b, identity, w_next, relu, nacc):
    n = dinv_b.shape[0]
    grid = n // BN
    have_res = identity is not None
    have_mm = w_next is not None

    def body(*refs):
        i = 0
        sp_ref = refs[i]; i += 1
        hs_ref = refs[i]; i += 1
        dinv_ref = refs[i]; i += 1
        b_ref = refs[i]; i += 1
        id_ref = None
        w_ref = None
        if have_res:
            id_ref = refs[i]; i += 1
        if have_mm:
            w_ref = refs[i]; i += 1
        h_ref = refs[i]; i += 1
        hs2_ref = refs[i] if have_mm else None

        dinv = dinv_ref[...]
        agg = dinv * (sp_ref[0] + sp_ref[1] + hs_ref[...]) + b_ref[...]
        if have_res:
            agg = agg + id_ref[...]
        h = jnp.maximum(agg, 0.0) if relu else agg
        h_ref[...] = h
        if have_mm:
            m = jnp.dot(h, w_ref[...], preferred_element_type=jnp.float32)
            hs2_ref[...] = dinv * m

    in_specs = [
        pl.BlockSpec((NC, BN, D), lambda i: (0, i, 0)),
        pl.BlockSpec((BN, D), lambda i: (i, 0)),
        pl.BlockSpec((BN, D), lambda i: (i, 0)),
        pl.BlockSpec((1, D), lambda i: (0, 0)),
    ]
    args = [sparts, hs, dinv_b, b.reshape(1, D)]
    if have_res:
        in_specs.append(pl.BlockSpec((BN, D), lambda i: (i, 0)))
        args.append(identity)
    if have_mm:
        in_specs.append(pl.BlockSpec((D, D), lambda i: (0, 0)))
        args.append(w_next)

    out_specs = [pl.BlockSpec((BN, D), lambda i: (i, 0))]
    out_shape = [jax.ShapeDtypeStruct((n, D), jnp.float32)]
    if have_mm:
        out_specs.append(pl.BlockSpec((BN, D), lambda i: (i, 0)))
        out_shape.append(jax.ShapeDtypeStruct((nacc, D), jnp.float32))

    res = pl.pallas_call(
        body,
        grid=(grid,),
        in_specs=in_specs,
        out_specs=out_specs,
        out_shape=out_shape,
    )(*args)
    return res if have_mm else (res[0], None)


def kernel(x, edge_index, W0, b0, W1, b1, W2, b2, W3, b3):
    n, d = x.shape
    e = edge_index.shape[1]
    assert n == N and d == D and n % BN == 0

    nacc = -(-(n + 1) // (NS * CH)) * (NS * CH)
    groups = -(-e // (NW * IG * CH))
    ep = groups * IG * CH * NW

    src_r = jnp.pad(edge_index[0], (0, ep - e)).reshape(NW, groups, IG, CH)
    dst_r = jnp.pad(edge_index[1], (0, ep - e), constant_values=n).reshape(
        NW, groups, IG, CH
    )

    degp = _sc_degree(nacc, groups)(dst_r)
    dinv_b, hs = _tc_first(x, W0, degp, nacc)

    agg = _sc_aggregate(nacc, groups)
    Ws = [W1, W2, W3, None]
    bs = [b0, b1, b2, b3]
    h_prev = None
    for i in range(4):
        sparts = agg(hs, src_r, dst_r)
        identity = h_prev if i in (1, 2) else None
        h, hs_next = _tc_combine(
            sparts, hs, dinv_b, bs[i], identity, Ws[i], relu=(i < 3), nacc=nacc
        )
        h_prev = h
        hs = hs_next
    return h_prev


# speedup vs baseline: 20.8593x; 1.0034x over previous
import functools

import jax
import jax.numpy as jnp
from jax import lax
from jax.experimental import pallas as pl
from jax.experimental.pallas import tpu as pltpu
from jax.experimental.pallas import tpu_sc as plsc

N = 10000
D = 128
NC = 2
NS = 16
NW = NC * NS
CH = 128
BN = 1000


def _mesh():
    return plsc.VectorSubcoreMesh(core_axis_name="c", subcore_axis_name="s")


IG = 20


DW = 16


def _sc_degree(nacc, groups):
    mesh = _mesh()
    rpt = nacc // NS

    @functools.partial(
        pl.kernel,
        mesh=mesh,
        out_type=jax.ShapeDtypeStruct((NC, nacc, DW), jnp.float32),
        compiler_params=pltpu.CompilerParams(use_tc_tiling_on_sc=False),
        scratch_types=[
            pltpu.VMEM((IG, CH), jnp.int32),
            pltpu.VMEM((CH, DW), jnp.float32),
            pltpu.VMEM_SHARED((nacc, DW), jnp.float32),
        ],
    )
    def deg_kernel(dst_hbm, out_hbm, dstv, ones, acc):
        c = lax.axis_index("c")
        s = lax.axis_index("s")
        wid = c * NS + s
        o16 = jnp.ones((16,), jnp.float32)
        z16 = jnp.zeros((16,), jnp.float32)

        def zfill(i, _):
            for j in range(DW // 16):
                ones[i, pl.ds(j * 16, 16)] = z16
            return 0

        def zcopy(k, _):
            pltpu.sync_copy(ones, acc.at[pl.ds(s * rpt + k * CH, CH)])
            return 0

        def fill(i, _):
            for j in range(DW // 16):
                ones[i, pl.ds(j * 16, 16)] = o16
            return 0

        lax.fori_loop(0, CH, zfill, 0)
        lax.fori_loop(0, rpt // CH, zcopy, 0)
        lax.fori_loop(0, CH, fill, 0)
        plsc.subcore_barrier()

        def body(g, _):
            pltpu.sync_copy(dst_hbm.at[wid, g], dstv)
            for k in range(IG):
                pltpu.sync_copy(ones, acc.at[dstv.at[k]], add=True)
            return 0

        lax.fori_loop(0, groups, body, 0)
        plsc.subcore_barrier()
        pltpu.sync_copy(
            acc.at[pl.ds(s * rpt, rpt)], out_hbm.at[c, pl.ds(s * rpt, rpt)]
        )

    return deg_kernel


HD = D // 2


def _sc_aggregate(nacc, groups):
    mesh = _mesh()
    rpt = nacc // NS

    @functools.partial(
        pl.kernel,
        mesh=mesh,
        out_type=jax.ShapeDtypeStruct((NC, nacc, D), jnp.float32),
        compiler_params=pltpu.CompilerParams(use_tc_tiling_on_sc=False),
        scratch_types=[
            pltpu.VMEM((2, 2, IG, CH), jnp.int32),
            pltpu.VMEM((4, CH, HD), jnp.float32),
            pltpu.VMEM_SHARED((nacc, HD), jnp.float32),
            pltpu.VMEM_SHARED((nacc, HD), jnp.float32),
            pltpu.SemaphoreType.DMA,
            ((pltpu.SemaphoreType.DMA,) * 4),
            ((pltpu.SemaphoreType.DMA,) * 4),
        ],
    )
    def agg_kernel(hs_hbm, src_hbm, dst_hbm, out_hbm, ibuf, rbuf, tbl, acc, isem, gsems, ssems):
        c = lax.axis_index("c")
        s = lax.axis_index("s")
        wid = c * NS + s
        z16 = jnp.zeros((16,), jnp.float32)

        for p in range(2):
            pltpu.sync_copy(
                hs_hbm.at[pl.ds(s * rpt, rpt), pl.ds(p * HD, HD)],
                tbl.at[pl.ds(s * rpt, rpt)],
            )

            @pl.when(c == 0)
            def _():
                pltpu.sync_copy(
                    hs_hbm.at[pl.ds(s * rpt, rpt), pl.ds(p * HD, HD)],
                    acc.at[pl.ds(s * rpt, rpt)],
                )

            @pl.when(c != 0)
            def _():
                def zfill(i, _):
                    for j in range(HD // 16):
                        rbuf[0, i, pl.ds(j * 16, 16)] = z16
                    return 0

                lax.fori_loop(0, CH, zfill, 0)

                def zcopy(k, _):
                    pltpu.sync_copy(rbuf.at[0], acc.at[pl.ds(s * rpt + k * CH, CH)])
                    return 0

                lax.fori_loop(0, rpt // CH, zcopy, 0)

            plsc.subcore_barrier()

            pltpu.async_copy(src_hbm.at[wid, 0], ibuf.at[0, 0], isem)
            pltpu.async_copy(dst_hbm.at[wid, 0], ibuf.at[0, 1], isem)

            def group_body(g, _):
                q = g % 2
                ibs = ibuf.at[q, 0]
                ibd = ibuf.at[q, 1]
                pltpu.make_async_copy(src_hbm.at[wid, g], ibs, isem).wait()
                pltpu.make_async_copy(dst_hbm.at[wid, g], ibd, isem).wait()

                @pl.when(g + 1 < groups)
                def _():
                    pltpu.async_copy(src_hbm.at[wid, g + 1], ibuf.at[1 - q, 0], isem)
                    pltpu.async_copy(dst_hbm.at[wid, g + 1], ibuf.at[1 - q, 1], isem)

                pltpu.async_copy(tbl.at[ibs.at[0]], rbuf.at[0], gsems[0])
                for k in range(IG):
                    j = k % 4
                    if k >= 3:
                        pltpu.make_async_copy(
                            rbuf.at[(k - 3) % 4], acc.at[ibd.at[k - 3]],
                            ssems[(k - 3) % 4],
                        ).wait()
                    if k + 1 < IG:
                        pltpu.async_copy(
                            tbl.at[ibs.at[k + 1]], rbuf.at[(k + 1) % 4],
                            gsems[(k + 1) % 4],
                        )
                    pltpu.make_async_copy(
                        tbl.at[ibs.at[k]], rbuf.at[j], gsems[j]
                    ).wait()
                    pltpu.async_copy(rbuf.at[j], acc.at[ibd.at[k]], ssems[j], add=True)
                for k in range(IG - 3, IG):
                    pltpu.make_async_copy(
                        rbuf.at[k % 4], acc.at[ibd.at[k]], ssems[k % 4]
                    ).wait()
                return 0

            lax.fori_loop(0, groups, group_body, 0)
            plsc.subcore_barrier()
            pltpu.sync_copy(
                acc.at[pl.ds(s * rpt, rpt)],
                out_hbm.at[c, pl.ds(s * rpt, rpt), pl.ds(p * HD, HD)],
            )
            if p == 0:
                plsc.subcore_barrier()

    return agg_kernel


def _tc_first(x, w, degp, nacc):
    n = x.shape[0]
    grid = n // BN

    def body(x_ref, w_ref, dp_ref, dinv_ref, hs_ref):
        deg = 1.0 + dp_ref[0, :, 0] + dp_ref[1, :, 0]
        dinv = lax.rsqrt(deg)[:, None]
        m = jnp.dot(x_ref[...], w_ref[...], preferred_element_type=jnp.float32)
        dinv_ref[...] = jnp.broadcast_to(dinv, (BN, D))
        hs_ref[...] = dinv * m

    return pl.pallas_call(
        body,
        grid=(grid,),
        in_specs=[
            pl.BlockSpec((BN, D), lambda i: (i, 0)),
            pl.BlockSpec((D, D), lambda i: (0, 0)),
            pl.BlockSpec((NC, BN, DW), lambda i: (0, i, 0)),
        ],
        out_specs=[
            pl.BlockSpec((BN, D), lambda i: (i, 0)),
            pl.BlockSpec((BN, D), lambda i: (i, 0)),
        ],
        out_shape=[
            jax.ShapeDtypeStruct((n, D), jnp.float32),
            jax.ShapeDtypeStruct((nacc, D), jnp.float32),
        ],
    )(x, w, degp)


def _tc_combine(sparts, dinv_b, b, identity, w_next, relu, nacc):
    n = dinv_b.shape[0]
    grid = n // BN
    have_res = identity is not None
    have_mm = w_next is not None

    def body(*refs):
        i = 0
        sp_ref = refs[i]; i += 1
        dinv_ref = refs[i]; i += 1
        b_ref = refs[i]; i += 1
        id_ref = None
        w_ref = None
        if have_res:
            id_ref = refs[i]; i += 1
        if have_mm:
            w_ref = refs[i]; i += 1
        h_ref = refs[i]; i += 1
        hs2_ref = refs[i] if have_mm else None

        dinv = dinv_ref[...]
        agg = dinv * (sp_ref[0] + sp_ref[1]) + b_ref[...]
        if have_res:
            agg = agg + id_ref[...]
        h = jnp.maximum(agg, 0.0) if relu else agg
        h_ref[...] = h
        if have_mm:
            m = jnp.dot(h, w_ref[...], preferred_element_type=jnp.float32)
            hs2_ref[...] = dinv * m

    in_specs = [
        pl.BlockSpec((NC, BN, D), lambda i: (0, i, 0)),
        pl.BlockSpec((BN, D), lambda i: (i, 0)),
        pl.BlockSpec((1, D), lambda i: (0, 0)),
    ]
    args = [sparts, dinv_b, b.reshape(1, D)]
    if have_res:
        in_specs.append(pl.BlockSpec((BN, D), lambda i: (i, 0)))
        args.append(identity)
    if have_mm:
        in_specs.append(pl.BlockSpec((D, D), lambda i: (0, 0)))
        args.append(w_next)

    out_specs = [pl.BlockSpec((BN, D), lambda i: (i, 0))]
    out_shape = [jax.ShapeDtypeStruct((n, D), jnp.float32)]
    if have_mm:
        out_specs.append(pl.BlockSpec((BN, D), lambda i: (i, 0)))
        out_shape.append(jax.ShapeDtypeStruct((nacc, D), jnp.float32))

    res = pl.pallas_call(
        body,
        grid=(grid,),
        in_specs=in_specs,
        out_specs=out_specs,
        out_shape=out_shape,
    )(*args)
    return res if have_mm else (res[0], None)


def kernel(x, edge_index, W0, b0, W1, b1, W2, b2, W3, b3):
    n, d = x.shape
    e = edge_index.shape[1]
    assert n == N and d == D and n % BN == 0

    nacc = -(-(n + 1) // (NS * CH)) * (NS * CH)
    groups = -(-e // (NW * IG * CH))
    ep = groups * IG * CH * NW

    src_r = jnp.pad(edge_index[0], (0, ep - e)).reshape(NW, groups, IG, CH)
    dst_r = jnp.pad(edge_index[1], (0, ep - e), constant_values=n).reshape(
        NW, groups, IG, CH
    )

    degp = _sc_degree(nacc, groups)(dst_r)
    dinv_b, hs = _tc_first(x, W0, degp, nacc)

    agg = _sc_aggregate(nacc, groups)
    Ws = [W1, W2, W3, None]
    bs = [b0, b1, b2, b3]
    h_prev = None
    for i in range(4):
        sparts = agg(hs, src_r, dst_r)
        identity = h_prev if i in (1, 2) else None
        h, hs_next = _tc_combine(
            sparts, dinv_b, bs[i], identity, Ws[i], relu=(i < 3), nacc=nacc
        )
        h_prev = h
        hs = hs_next
    return h_prev
